# baseline (speedup 1.0000x reference)
import functools

import jax
import jax.numpy as jnp
from jax import lax
from jax.experimental import pallas as pl
from jax.experimental.pallas import tpu as pltpu, tpu_sc as plsc

RO = 4.0
DILS = (1, 1, 1, 1, 2, 2, 4, 4)
N, P, C = 64, 128, 64
NP = N * P
H = W = 128
NW = 32


def _prep_body(in_ref, pyin_ref, ch_ref, idx_ref, w_ref, *, first):
    base = in_ref[...] * (1.0 / RO)
    if first:
        pos = jnp.clip(base, 0.0, W - 1.0)
    else:
        pos = base
    pyin_ref[...] = pos * RO
    can3 = base.reshape(2, N, P)
    mn = jnp.min(can3, axis=2, keepdims=True)
    ch_ref[...] = ((can3 - mn) * RO).reshape(2, NP)

    px = pos[0:1, :] - 0.5
    py = pos[1:2, :] - 0.5
    x0 = jnp.floor(px)
    y0 = jnp.floor(py)
    wx1 = px - x0
    wx0 = 1.0 - wx1
    wy1 = py - y0
    wy0 = 1.0 - wy1
    xr = jnp.clip(x0, 0.0, W - 2.0)
    ws0 = wx0 * (xr == x0) + wx1 * (xr == x0 + 1.0)
    ws1 = wx0 * (xr + 1.0 == x0) + wx1 * (xr == x0)
    wyv0 = wy0 * ((y0 >= 0.0) & (y0 <= H - 1.0))
    wyv1 = wy1 * ((y0 >= -1.0) & (y0 <= H - 2.0))
    yc0 = jnp.clip(y0, 0.0, H - 1.0).astype(jnp.int32)
    yc1 = jnp.clip(y0 + 1.0, 0.0, H - 1.0).astype(jnp.int32)
    xri = xr.astype(jnp.int32)
    idx_ref[...] = jnp.concatenate([yc0 * W + xri, yc1 * W + xri], axis=0)
    w_ref[...] = jnp.concatenate([wyv0 * ws0, wyv0 * ws1,
                                  wyv1 * ws0, wyv1 * ws1], axis=0)


def _snake_body(G_ref, wexp_ref, rest_ref, pyin_ref, Wc_ref, b_ref, g_ref,
                be_ref, fp_ref, fusb_ref, p0w_ref, p0b_ref, p1w_ref, p1b_ref,
                p2w_ref, p2b_ref, py_ref):
    M = G_ref[...] * wexp_ref[...]
    feat = (M[:, 0:C] + M[:, C:2 * C]) + (M[:, 2 * C:3 * C] + M[:, 3 * C:4 * C])
    xb = jnp.concatenate([feat.astype(jnp.bfloat16), rest_ref[...]], axis=1)

    fp_acc = jnp.zeros((NP, 512), jnp.float32)
    x = None
    for l in range(8):
        d = DILS[l]
        x3 = xb.reshape(N, P, 128)
        xd = jnp.concatenate([x3[:, d:, :], x3[:, :d, :]], axis=1)
        Pc = jnp.concatenate([xb, xd.reshape(NP, 128)], axis=1)
        Y3 = jnp.dot(Pc, Wc_ref[l],
                     preferred_element_type=jnp.float32).reshape(N, P, 640)
        acc = Y3[:, :, 256:384]
        for j in (0, 1, 3, 4):
            s = ((2 * j - 4) * d) % P
            blk = Y3[:, :, j * 128:(j + 1) * 128]
            acc = acc + jnp.concatenate([blk[:, s:, :], blk[:, :s, :]], axis=1)
        y = jnp.maximum(acc.reshape(NP, 128) + b_ref[l][None, :], 0.0)
        m = jnp.mean(y, axis=0, keepdims=True)
        v = jnp.mean((y - m) ** 2, axis=0, keepdims=True)
        y = (y - m) * lax.rsqrt(v + 1e-5) * g_ref[l][None, :] + be_ref[l][None, :]
        x = y if l == 0 else x + y
        xb = x.astype(jnp.bfloat16)
        if l % 2 == 0:
            x_even = xb
        else:
            fp_acc += jnp.dot(jnp.concatenate([x_even, xb], axis=1),
                              fp_ref[l // 2], preferred_element_type=jnp.float32)

    fused_acc = fp_acc[:, 0:256]
    p0_acc = fp_acc[:, 256:512]
    gmax = (jnp.max(fused_acc.reshape(N, P, 256), axis=1, keepdims=False)
            + fusb_ref[...])
    t_gs = jnp.dot(gmax.astype(jnp.bfloat16), p0w_ref[...],
                   preferred_element_type=jnp.float32)
    t_gs = jnp.broadcast_to(t_gs[:, None, :], (N, P, 256)).reshape(NP, 256)
    t = jnp.maximum(t_gs + p0_acc + p0b_ref[...], 0.0)
    t = jnp.maximum(jnp.dot(t.astype(jnp.bfloat16), p1w_ref[...],
                            preferred_element_type=jnp.float32)
                    + p1b_ref[...], 0.0)
    off = jnp.dot(t.astype(jnp.bfloat16), p2w_ref[...],
                  preferred_element_type=jnp.float32) + p2b_ref[...]
    py_ref[...] = pyin_ref[...] + off


@functools.cache
def _gather_call():
    mesh = plsc.VectorSubcoreMesh(core_axis_name="c", subcore_axis_name="s")

    @functools.partial(
        pl.kernel, mesh=mesh,
        out_type=jax.ShapeDtypeStruct((NW, 4, 128, 2 * C), jnp.float32),
        scratch_types=[pltpu.VMEM((4, 128), jnp.int32),
                       pltpu.VMEM((4, 128, 2 * C), jnp.float32),
                       pltpu.SemaphoreType.DMA],
    )
    def gather_k(table_hbm, idx_hbm, out_hbm, idx_v, rows_v, sem):
        wid = lax.axis_index("s") * 2 + lax.axis_index("c")
        pltpu.sync_copy(idx_hbm.at[wid], idx_v)
        handles = [pltpu.async_copy(table_hbm.at[idx_v.at[j]], rows_v.at[j], sem)
                   for j in range(4)]
        for h in handles:
            h.wait()
        pltpu.sync_copy(rows_v, out_hbm.at[wid])

    return gather_k


_PREP_OUT = (jax.ShapeDtypeStruct((2, NP), jnp.float32),
             jax.ShapeDtypeStruct((2, NP), jnp.float32),
             jax.ShapeDtypeStruct((2, NP), jnp.int32),
             jax.ShapeDtypeStruct((4, NP), jnp.float32))


@functools.cache
def _prep_call(first):
    return pl.pallas_call(functools.partial(_prep_body, first=first),
                          out_shape=_PREP_OUT)


@functools.cache
def _snake_call():
    return pl.pallas_call(
        _snake_body,
        out_shape=jax.ShapeDtypeStruct((NP, 128), jnp.float32))


def kernel(poly_coarse, cnn_feature, head_w, head_b, head_g, head_be, res_w, res_b,
           res_g, res_be, fus_w, fus_b, p0_w, p0_b, p1_w, p1_b, p2_w, p2_b, py_ind):
    S = head_w.shape[0]
    feat3 = jnp.transpose(cnn_feature[0], (1, 2, 0))
    feat_n = jnp.concatenate([feat3[:, 1:], feat3[:, -1:]], axis=1)
    table = jnp.concatenate([feat3, feat_n], axis=2).reshape(H * W, 2 * C)

    head_wt = jnp.transpose(head_w, (0, 3, 2, 1))
    head_wt = jnp.pad(head_wt, ((0, 0), (0, 0), (0, 128 - 66), (0, 0)))
    head_wt = head_wt.reshape(S, 1, 9 * 128, 128)
    res_wt = jnp.transpose(res_w, (0, 1, 4, 3, 2)).reshape(S, 7, 9 * 128, 128)
    Wall = jnp.concatenate([head_wt, res_wt], axis=1)
    Wt10 = jnp.pad(Wall.reshape(S, 8, 9, 128, 128),
                   ((0, 0), (0, 0), (0, 1), (0, 0), (0, 0)))
    Wc = jnp.transpose(Wt10.reshape(S, 8, 5, 256, 128),
                       (0, 1, 3, 2, 4)).reshape(S, 8, 256, 640).astype(jnp.bfloat16)
    b_all = jnp.concatenate([head_b[:, None], res_b], axis=1)
    g_all = jnp.concatenate([head_g[:, None], res_g], axis=1)
    be_all = jnp.concatenate([head_be[:, None], res_be], axis=1)
    fus_wt = jnp.transpose(fus_w, (0, 2, 1))
    p0_wt = jnp.transpose(p0_w, (0, 2, 1))
    fp_wb = jnp.concatenate([fus_wt.reshape(S, 8, 128, 256),
                             p0_wt[:, 256:, :].reshape(S, 8, 128, 256)],
                            axis=3).reshape(S, 4, 256, 512).astype(jnp.bfloat16)
    p0_gs = p0_wt[:, 0:256, :].astype(jnp.bfloat16)
    p1_wb = jnp.transpose(p1_w, (0, 2, 1)).astype(jnp.bfloat16)
    p2_wb = jnp.pad(jnp.transpose(p2_w, (0, 2, 1)),
                    ((0, 0), (0, 0), (0, 126))).astype(jnp.bfloat16)
    p2_bp = jnp.pad(p2_b, ((0, 0), (0, 126)))

    gather = _gather_call()
    snake = _snake_call()
    cur_t = jnp.transpose(poly_coarse.reshape(NP, 2))
    pys = []
    for s in range(3):
        pyin_t, ch_t, idx_t, w4_t = _prep_call(s == 0)(cur_t)
        idx_sc = jnp.transpose(idx_t).reshape(NW, 4, 128)
        wexp = jnp.broadcast_to(jnp.transpose(w4_t)[:, :, None],
                                (NP, 4, C)).reshape(NP, 4 * C).astype(jnp.bfloat16)
        rest = jnp.concatenate(
            [jnp.transpose(ch_t), jnp.zeros((NP, 128 - C - 2), jnp.float32)],
            axis=1).astype(jnp.bfloat16)
        pyin_pad = jnp.pad(jnp.transpose(pyin_t), ((0, 0), (0, 126)))

        G = gather(table, idx_sc)
        G = G.reshape(NP, 4 * C)
        py_pad = snake(
            G, wexp, rest, pyin_pad, Wc[s], b_all[s], g_all[s], be_all[s],
            fp_wb[s], fus_b[s][None, :], p0_gs[s], p0_b[s][None, :],
            p1_wb[s], p1_b[s][None, :], p2_wb[s], p2_bp[s][None, :])
        py = py_pad[:, :2]
        pys.append(py.reshape(N, P, 2))
        cur_t = jnp.transpose(py)
    return jnp.stack(pys)


# speedup vs baseline: 1.0189x; 1.0189x over previous
import functools

import jax
import jax.numpy as jnp
from jax import lax
from jax.experimental import pallas as pl
from jax.experimental.pallas import tpu as pltpu, tpu_sc as plsc

RO = 4.0
DILS = (1, 1, 1, 1, 2, 2, 4, 4)
N, P, C = 64, 128, 64
NP = N * P
H = W = 128
NW = 32


def _prep_body(in_ref, pyin_ref, ch_ref, idx_ref, w_ref, *, first):
    base = in_ref[...] * (1.0 / RO)
    if first:
        pos = jnp.clip(base, 0.0, W - 1.0)
    else:
        pos = base
    pyin_ref[...] = pos * RO
    can3 = base.reshape(2, N, P)
    mn = jnp.min(can3, axis=2, keepdims=True)
    ch_ref[...] = ((can3 - mn) * RO).reshape(2, NP)

    px = pos[0:1, :] - 0.5
    py = pos[1:2, :] - 0.5
    x0 = jnp.floor(px)
    y0 = jnp.floor(py)
    wx1 = px - x0
    wx0 = 1.0 - wx1
    wy1 = py - y0
    wy0 = 1.0 - wy1
    xr = jnp.clip(x0, 0.0, W - 2.0)
    ws0 = wx0 * (xr == x0) + wx1 * (xr == x0 + 1.0)
    ws1 = wx0 * (xr + 1.0 == x0) + wx1 * (xr == x0)
    wyv0 = wy0 * ((y0 >= 0.0) & (y0 <= H - 1.0))
    wyv1 = wy1 * ((y0 >= -1.0) & (y0 <= H - 2.0))
    yc0 = jnp.clip(y0, 0.0, H - 1.0).astype(jnp.int32)
    yc1 = jnp.clip(y0 + 1.0, 0.0, H - 1.0).astype(jnp.int32)
    xri = xr.astype(jnp.int32)
    idx_ref[...] = jnp.concatenate([yc0 * W + xri, yc1 * W + xri], axis=0)
    w_ref[...] = jnp.concatenate([wyv0 * ws0, wyv0 * ws1,
                                  wyv1 * ws0, wyv1 * ws1], axis=0)


def _snake_body(G_ref, wexp_ref, rest_ref, pyin_ref, Wc_ref, b_ref, g_ref,
                be_ref, fp_ref, fusb_ref, p0w_ref, p0b_ref, p1w_ref, p1b_ref,
                p2w_ref, p2b_ref, py_ref):
    M = G_ref[...] * wexp_ref[...]
    feat = (M[:, 0:C] + M[:, C:2 * C]) + (M[:, 2 * C:3 * C] + M[:, 3 * C:4 * C])
    xb = jnp.concatenate([feat, rest_ref[...]], axis=1)

    fp_acc = jnp.zeros((NP, 512), jnp.float32)
    x = None
    for l in range(8):
        d = DILS[l]
        x3 = xb.reshape(N, P, 128)
        xd = jnp.concatenate([x3[:, d:, :], x3[:, :d, :]], axis=1)
        Pc = jnp.concatenate([xb, xd.reshape(NP, 128)], axis=1)
        Y3 = jnp.dot(Pc, Wc_ref[l],
                     preferred_element_type=jnp.float32).reshape(N, P, 640)
        acc = Y3[:, :, 256:384]
        for j in (0, 1, 3, 4):
            s = ((2 * j - 4) * d) % P
            blk = Y3[:, :, j * 128:(j + 1) * 128]
            acc = acc + jnp.concatenate([blk[:, s:, :], blk[:, :s, :]], axis=1)
        y = jnp.maximum(acc.reshape(NP, 128) + b_ref[l][None, :], 0.0)
        m = jnp.mean(y, axis=0, keepdims=True)
        v = jnp.mean((y - m) ** 2, axis=0, keepdims=True)
        y = (y - m) * lax.rsqrt(v + 1e-5) * g_ref[l][None, :] + be_ref[l][None, :]
        x = y if l == 0 else x + y
        xb = x.astype(jnp.bfloat16)
        if l % 2 == 0:
            x_even = xb
        else:
            fp_acc += jnp.dot(jnp.concatenate([x_even, xb], axis=1),
                              fp_ref[l // 2], preferred_element_type=jnp.float32)

    fused_acc = fp_acc[:, 0:256]
    p0_acc = fp_acc[:, 256:512]
    gmax = (jnp.max(fused_acc.reshape(N, P, 256), axis=1, keepdims=False)
            + fusb_ref[...])
    t_gs = jnp.dot(gmax.astype(jnp.bfloat16), p0w_ref[...],
                   preferred_element_type=jnp.float32)
    t_gs = jnp.broadcast_to(t_gs[:, None, :], (N, P, 256)).reshape(NP, 256)
    t = jnp.maximum(t_gs + p0_acc + p0b_ref[...], 0.0)
    t = jnp.maximum(jnp.dot(t.astype(jnp.bfloat16), p1w_ref[...],
                            preferred_element_type=jnp.float32)
                    + p1b_ref[...], 0.0)
    off = jnp.dot(t.astype(jnp.bfloat16), p2w_ref[...],
                  preferred_element_type=jnp.float32) + p2b_ref[...]
    py_ref[...] = pyin_ref[...] + off


@functools.cache
def _gather_call():
    mesh = plsc.VectorSubcoreMesh(core_axis_name="c", subcore_axis_name="s")

    @functools.partial(
        pl.kernel, mesh=mesh,
        out_type=jax.ShapeDtypeStruct((NW, 4, 128, 2 * C), jnp.float32),
        scratch_types=[pltpu.VMEM((4, 128), jnp.int32),
                       pltpu.VMEM((4, 128, 2 * C), jnp.float32),
                       pltpu.SemaphoreType.DMA],
    )
    def gather_k(table_hbm, idx_hbm, out_hbm, idx_v, rows_v, sem):
        wid = lax.axis_index("s") * 2 + lax.axis_index("c")
        pltpu.sync_copy(idx_hbm.at[wid], idx_v)
        handles = [pltpu.async_copy(table_hbm.at[idx_v.at[j]], rows_v.at[j], sem)
                   for j in range(4)]
        for h in handles:
            h.wait()
        pltpu.sync_copy(rows_v, out_hbm.at[wid])

    return gather_k


_PREP_OUT = (jax.ShapeDtypeStruct((2, NP), jnp.float32),
             jax.ShapeDtypeStruct((2, NP), jnp.float32),
             jax.ShapeDtypeStruct((2, NP), jnp.int32),
             jax.ShapeDtypeStruct((4, NP), jnp.float32))


@functools.cache
def _prep_call(first):
    return pl.pallas_call(functools.partial(_prep_body, first=first),
                          out_shape=_PREP_OUT)


@functools.cache
def _snake_call():
    return pl.pallas_call(
        _snake_body,
        out_shape=jax.ShapeDtypeStruct((NP, 128), jnp.float32))


def kernel(poly_coarse, cnn_feature, head_w, head_b, head_g, head_be, res_w, res_b,
           res_g, res_be, fus_w, fus_b, p0_w, p0_b, p1_w, p1_b, p2_w, p2_b, py_ind):
    S = head_w.shape[0]
    feat3 = jnp.transpose(cnn_feature[0], (1, 2, 0))
    feat_n = jnp.concatenate([feat3[:, 1:], feat3[:, -1:]], axis=1)
    table = jnp.concatenate([feat3, feat_n], axis=2).reshape(H * W, 2 * C)

    head_wt = jnp.transpose(head_w, (0, 3, 2, 1))
    head_wt = jnp.pad(head_wt, ((0, 0), (0, 0), (0, 128 - 66), (0, 0)))
    head_wt = head_wt.reshape(S, 1, 9 * 128, 128)
    res_wt = jnp.transpose(res_w, (0, 1, 4, 3, 2)).reshape(S, 7, 9 * 128, 128)
    Wall = jnp.concatenate([head_wt, res_wt], axis=1)
    Wt10 = jnp.pad(Wall.reshape(S, 8, 9, 128, 128),
                   ((0, 0), (0, 0), (0, 1), (0, 0), (0, 0)))
    Wc = jnp.transpose(Wt10.reshape(S, 8, 5, 256, 128),
                       (0, 1, 3, 2, 4)).reshape(S, 8, 256, 640).astype(jnp.bfloat16)
    b_all = jnp.concatenate([head_b[:, None], res_b], axis=1)
    g_all = jnp.concatenate([head_g[:, None], res_g], axis=1)
    be_all = jnp.concatenate([head_be[:, None], res_be], axis=1)
    fus_wt = jnp.transpose(fus_w, (0, 2, 1))
    p0_wt = jnp.transpose(p0_w, (0, 2, 1))
    fp_wb = jnp.concatenate([fus_wt.reshape(S, 8, 128, 256),
                             p0_wt[:, 256:, :].reshape(S, 8, 128, 256)],
                            axis=3).reshape(S, 4, 256, 512).astype(jnp.bfloat16)
    p0_gs = p0_wt[:, 0:256, :].astype(jnp.bfloat16)
    p1_wb = jnp.transpose(p1_w, (0, 2, 1)).astype(jnp.bfloat16)
    p2_wb = jnp.pad(jnp.transpose(p2_w, (0, 2, 1)),
                    ((0, 0), (0, 0), (0, 126))).astype(jnp.bfloat16)
    p2_bp = jnp.pad(p2_b, ((0, 0), (0, 126)))

    gather = _gather_call()
    snake = _snake_call()
    cur_t = jnp.transpose(poly_coarse.reshape(NP, 2))
    pys = []
    for s in range(3):
        pyin_t, ch_t, idx_t, w4_t = _prep_call(s == 0)(cur_t)
        idx_sc = jnp.transpose(idx_t).reshape(NW, 4, 128)
        wexp = jnp.broadcast_to(jnp.transpose(w4_t)[:, :, None],
                                (NP, 4, C)).reshape(NP, 4 * C).astype(jnp.bfloat16)
        rest = jnp.concatenate(
            [jnp.transpose(ch_t), jnp.zeros((NP, 128 - C - 2), jnp.float32)],
            axis=1).astype(jnp.bfloat16)
        pyin_pad = jnp.pad(jnp.transpose(pyin_t), ((0, 0), (0, 126)))

        G = gather(table, idx_sc)
        G = G.reshape(NP, 4 * C).astype(jnp.bfloat16)
        py_pad = snake(
            G, wexp, rest, pyin_pad, Wc[s], b_all[s], g_all[s], be_all[s],
            fp_wb[s], fus_b[s][None, :], p0_gs[s], p0_b[s][None, :],
            p1_wb[s], p1_b[s][None, :], p2_wb[s], p2_bp[s][None, :])
        py = py_pad[:, :2]
        pys.append(py.reshape(N, P, 2))
        cur_t = jnp.transpose(py)
    return jnp.stack(pys)


# speedup vs baseline: 1.0190x; 1.0001x over previous
import functools

import jax
import jax.numpy as jnp
from jax import lax
from jax.experimental import pallas as pl
from jax.experimental.pallas import tpu as pltpu, tpu_sc as plsc

RO = 4.0
DILS = (1, 1, 1, 1, 2, 2, 4, 4)
N, P, C = 64, 128, 64
NP = N * P
H = W = 128
NW = 32


def _prep_body(in_ref, pyin_ref, ch_ref, idx_ref, w_ref, *, first):
    base = in_ref[...] * (1.0 / RO)
    if first:
        pos = jnp.clip(base, 0.0, W - 1.0)
    else:
        pos = base
    pyin_ref[...] = pos * RO
    can3 = base.reshape(2, N, P)
    mn = jnp.min(can3, axis=2, keepdims=True)
    ch_ref[...] = ((can3 - mn) * RO).reshape(2, NP)

    px = pos[0:1, :] - 0.5
    py = pos[1:2, :] - 0.5
    x0 = jnp.floor(px)
    y0 = jnp.floor(py)
    wx1 = px - x0
    wx0 = 1.0 - wx1
    wy1 = py - y0
    wy0 = 1.0 - wy1
    xr = jnp.clip(x0, 0.0, W - 2.0)
    ws0 = wx0 * (xr == x0) + wx1 * (xr == x0 + 1.0)
    ws1 = wx0 * (xr + 1.0 == x0) + wx1 * (xr == x0)
    wyv0 = wy0 * ((y0 >= 0.0) & (y0 <= H - 1.0))
    wyv1 = wy1 * ((y0 >= -1.0) & (y0 <= H - 2.0))
    yc0 = jnp.clip(y0, 0.0, H - 1.0).astype(jnp.int32)
    yc1 = jnp.clip(y0 + 1.0, 0.0, H - 1.0).astype(jnp.int32)
    xri = xr.astype(jnp.int32)
    idx_ref[...] = jnp.concatenate([yc0 * W + xri, yc1 * W + xri], axis=0)
    w_ref[...] = jnp.concatenate([wyv0 * ws0, wyv0 * ws1,
                                  wyv1 * ws0, wyv1 * ws1], axis=0)


def _snake_body(G_ref, wexp_ref, rest_ref, pyin_ref, Wc_ref, b_ref, g_ref,
                be_ref, fp_ref, fusb_ref, p0w_ref, p0b_ref, p1w_ref, p1b_ref,
                p2w_ref, p2b_ref, py_ref):
    M = G_ref[...] * wexp_ref[...]
    feat = (M[:, 0:C] + M[:, C:2 * C]) + (M[:, 2 * C:3 * C] + M[:, 3 * C:4 * C])
    xb = jnp.concatenate([feat, rest_ref[...]], axis=1)

    fp_acc = jnp.zeros((NP, 512), jnp.float32)
    x = None
    for l in range(8):
        d = DILS[l]
        x3 = xb.reshape(N, P, 128)
        xd = jnp.concatenate([x3[:, d:, :], x3[:, :d, :]], axis=1)
        Pc = jnp.concatenate([xb, xd.reshape(NP, 128)], axis=1)
        Y3 = jnp.dot(Pc, Wc_ref[l],
                     preferred_element_type=jnp.float32).reshape(N, P, 640)
        acc = Y3[:, :, 256:384]
        for j in (0, 1, 3, 4):
            s = ((2 * j - 4) * d) % P
            blk = Y3[:, :, j * 128:(j + 1) * 128]
            acc = acc + jnp.concatenate([blk[:, s:, :], blk[:, :s, :]], axis=1)
        y = jnp.maximum(acc.reshape(NP, 128) + b_ref[l][None, :], 0.0)
        m = jnp.mean(y, axis=0, keepdims=True)
        v = jnp.mean((y - m) ** 2, axis=0, keepdims=True)
        y = (y - m) * lax.rsqrt(v + 1e-5) * g_ref[l][None, :] + be_ref[l][None, :]
        x = y if l == 0 else x + y
        xb = x.astype(jnp.bfloat16)
        if l % 2 == 0:
            x_even = xb
        else:
            fp_acc += jnp.dot(jnp.concatenate([x_even, xb], axis=1),
                              fp_ref[l // 2], preferred_element_type=jnp.float32)

    fused_acc = fp_acc[:, 0:256]
    p0_acc = fp_acc[:, 256:512]
    gmax = (jnp.max(fused_acc.reshape(N, P, 256), axis=1, keepdims=False)
            + fusb_ref[...])
    t_gs = jnp.dot(gmax.astype(jnp.bfloat16), p0w_ref[...],
                   preferred_element_type=jnp.float32)
    t_gs = jnp.broadcast_to(t_gs[:, None, :], (N, P, 256)).reshape(NP, 256)
    t = jnp.maximum(t_gs + p0_acc + p0b_ref[...], 0.0)
    t = jnp.maximum(jnp.dot(t.astype(jnp.bfloat16), p1w_ref[...],
                            preferred_element_type=jnp.float32)
                    + p1b_ref[...], 0.0)
    off = jnp.dot(t.astype(jnp.bfloat16), p2w_ref[...],
                  preferred_element_type=jnp.float32) + p2b_ref[...]
    py_ref[...] = pyin_ref[...] + off


@functools.cache
def _gather_call():
    mesh = plsc.VectorSubcoreMesh(core_axis_name="c", subcore_axis_name="s")

    @functools.partial(
        pl.kernel, mesh=mesh,
        out_type=jax.ShapeDtypeStruct((NW, 4, 128, 2 * C), jnp.float32),
        scratch_types=[pltpu.VMEM((4, 128), jnp.int32),
                       pltpu.VMEM((4, 128, 2 * C), jnp.float32),
                       pltpu.SemaphoreType.DMA],
    )
    def gather_k(table_hbm, idx_hbm, out_hbm, idx_v, rows_v, sem):
        wid = lax.axis_index("s") * 2 + lax.axis_index("c")
        pltpu.sync_copy(idx_hbm.at[wid], idx_v)
        handles = [pltpu.async_copy(table_hbm.at[idx_v.at[j]], rows_v.at[j], sem)
                   for j in range(4)]
        for h in handles:
            h.wait()
        pltpu.sync_copy(rows_v, out_hbm.at[wid])

    return gather_k


_PREP_OUT = (jax.ShapeDtypeStruct((2, NP), jnp.float32),
             jax.ShapeDtypeStruct((2, NP), jnp.float32),
             jax.ShapeDtypeStruct((2, NP), jnp.int32),
             jax.ShapeDtypeStruct((4, NP), jnp.float32))


@functools.cache
def _prep_call(first):
    return pl.pallas_call(functools.partial(_prep_body, first=first),
                          out_shape=_PREP_OUT)


@functools.cache
def _snake_call():
    return pl.pallas_call(
        _snake_body,
        out_shape=jax.ShapeDtypeStruct((NP, 8), jnp.float32))


def kernel(poly_coarse, cnn_feature, head_w, head_b, head_g, head_be, res_w, res_b,
           res_g, res_be, fus_w, fus_b, p0_w, p0_b, p1_w, p1_b, p2_w, p2_b, py_ind):
    S = head_w.shape[0]
    feat3 = jnp.transpose(cnn_feature[0], (1, 2, 0))
    feat_n = jnp.concatenate([feat3[:, 1:], feat3[:, -1:]], axis=1)
    table = jnp.concatenate([feat3, feat_n], axis=2).reshape(H * W, 2 * C)

    head_wt = jnp.transpose(head_w, (0, 3, 2, 1))
    head_wt = jnp.pad(head_wt, ((0, 0), (0, 0), (0, 128 - 66), (0, 0)))
    head_wt = head_wt.reshape(S, 1, 9 * 128, 128)
    res_wt = jnp.transpose(res_w, (0, 1, 4, 3, 2)).reshape(S, 7, 9 * 128, 128)
    Wall = jnp.concatenate([head_wt, res_wt], axis=1)
    Wt10 = jnp.pad(Wall.reshape(S, 8, 9, 128, 128),
                   ((0, 0), (0, 0), (0, 1), (0, 0), (0, 0)))
    Wc = jnp.transpose(Wt10.reshape(S, 8, 5, 256, 128),
                       (0, 1, 3, 2, 4)).reshape(S, 8, 256, 640).astype(jnp.bfloat16)
    b_all = jnp.concatenate([head_b[:, None], res_b], axis=1)
    g_all = jnp.concatenate([head_g[:, None], res_g], axis=1)
    be_all = jnp.concatenate([head_be[:, None], res_be], axis=1)
    fus_wt = jnp.transpose(fus_w, (0, 2, 1))
    p0_wt = jnp.transpose(p0_w, (0, 2, 1))
    fp_wb = jnp.concatenate([fus_wt.reshape(S, 8, 128, 256),
                             p0_wt[:, 256:, :].reshape(S, 8, 128, 256)],
                            axis=3).reshape(S, 4, 256, 512).astype(jnp.bfloat16)
    p0_gs = p0_wt[:, 0:256, :].astype(jnp.bfloat16)
    p1_wb = jnp.transpose(p1_w, (0, 2, 1)).astype(jnp.bfloat16)
    p2_wb = jnp.pad(jnp.transpose(p2_w, (0, 2, 1)),
                    ((0, 0), (0, 0), (0, 6))).astype(jnp.bfloat16)
    p2_bp = jnp.pad(p2_b, ((0, 0), (0, 6)))

    gather = _gather_call()
    snake = _snake_call()
    cur_t = jnp.transpose(poly_coarse.reshape(NP, 2))
    pys = []
    for s in range(3):
        pyin_t, ch_t, idx_t, w4_t = _prep_call(s == 0)(cur_t)
        idx_sc = jnp.transpose(idx_t).reshape(NW, 4, 128)
        wexp = jnp.broadcast_to(jnp.transpose(w4_t)[:, :, None],
                                (NP, 4, C)).reshape(NP, 4 * C).astype(jnp.bfloat16)
        rest = jnp.concatenate(
            [jnp.transpose(ch_t), jnp.zeros((NP, 128 - C - 2), jnp.float32)],
            axis=1).astype(jnp.bfloat16)
        pyin_pad = jnp.pad(jnp.transpose(pyin_t), ((0, 0), (0, 6)))

        G = gather(table, idx_sc)
        G = G.reshape(NP, 4 * C).astype(jnp.bfloat16)
        py_pad = snake(
            G, wexp, rest, pyin_pad, Wc[s], b_all[s], g_all[s], be_all[s],
            fp_wb[s], fus_b[s][None, :], p0_gs[s], p0_b[s][None, :],
            p1_wb[s], p1_b[s][None, :], p2_wb[s], p2_bp[s][None, :])
        py = py_pad[:, :2]
        pys.append(py.reshape(N, P, 2))
        cur_t = jnp.transpose(py)
    return jnp.stack(pys)
